# Initial kernel scaffold; baseline (speedup 1.0000x reference)
import functools

import jax
import jax.numpy as jnp
from jax import lax
from jax.experimental import pallas as pl
from jax.experimental.pallas import tpu as pltpu
from jax.experimental.pallas import tpu_sc as plsc

N_NODES = 10000
N_EDGES = 320000
D = 128
NCLS = 16
N_GRAPHS = 64

NC, NS = 2, 16
NW = NC * NS
NPAD = 10240
RPW = NPAD // NS
EPW = N_EDGES // NW
CH = 80
NCHUNK = EPW // CH

BR = 1024
GRID = NPAD // BR


def _sc_agg_wide(x, src, dst, zeros2d, zeros1d):
    mesh = plsc.VectorSubcoreMesh(core_axis_name="c", subcore_axis_name="s")

    @functools.partial(
        pl.kernel,
        out_type=(
            jax.ShapeDtypeStruct((NC, NPAD, D), jnp.float32),
            jax.ShapeDtypeStruct((NC, NPAD), jnp.float32),
        ),
        mesh=mesh,
        scratch_types=[
            pltpu.VMEM((1, CH), jnp.int32),
            pltpu.VMEM((1, CH), jnp.int32),
            pltpu.VMEM((CH, D), jnp.float32),
            pltpu.VMEM((CH,), jnp.float32),
            pltpu.VMEM_SHARED((NPAD, D), jnp.float32),
            pltpu.VMEM_SHARED((NPAD,), jnp.float32),
            pltpu.SemaphoreType.DMA,
        ],
    )
    def k(x_hbm, src_hbm, dst_hbm, z2_hbm, z1_hbm, acc_out, cnt_out,
          si, di, rows, ones, acc_sh, cnt_sh, sem):
        cid = lax.axis_index("c")
        sid = lax.axis_index("s")
        wid = cid * NS + sid

        r0 = sid * RPW
        pltpu.sync_copy(z2_hbm.at[pl.ds(r0, RPW), :], acc_sh.at[pl.ds(r0, RPW), :])
        pltpu.sync_copy(z1_hbm.at[pl.ds(r0, RPW)], cnt_sh.at[pl.ds(r0, RPW)])

        @pl.loop(0, CH, step=16)
        def _(i):
            ones[pl.ds(i, 16)] = jnp.ones((16,), jnp.float32)

        plsc.subcore_barrier()

        base = wid * EPW

        @pl.loop(0, NCHUNK)
        def _(j):
            off = base + j * CH
            pltpu.sync_copy(src_hbm.at[pl.ds(off, CH)], si.at[0])
            pltpu.sync_copy(dst_hbm.at[pl.ds(off, CH)], di.at[0])
            pltpu.async_copy(x_hbm.at[si.at[0]], rows, sem).wait()
            pltpu.sync_copy(rows, acc_sh.at[di.at[0]], add=True)
            pltpu.sync_copy(ones, cnt_sh.at[di.at[0]], add=True)

        plsc.subcore_barrier()

        pltpu.sync_copy(acc_sh.at[pl.ds(r0, RPW), :],
                        acc_out.at[cid, pl.ds(r0, RPW), :])
        pltpu.sync_copy(cnt_sh.at[pl.ds(r0, RPW)],
                        cnt_out.at[cid, pl.ds(r0, RPW)])

    return k(x, src, dst, zeros2d, zeros1d)


def _sc_agg_narrow(hl, src, dst, zeros16):
    mesh = plsc.VectorSubcoreMesh(core_axis_name="c", subcore_axis_name="s")

    @functools.partial(
        pl.kernel,
        out_type=jax.ShapeDtypeStruct((NC, NPAD, NCLS), jnp.float32),
        mesh=mesh,
        scratch_types=[
            pltpu.VMEM((1, CH), jnp.int32),
            pltpu.VMEM((1, CH), jnp.int32),
            pltpu.VMEM((CH, NCLS), jnp.float32),
            pltpu.VMEM_SHARED((NPAD, NCLS), jnp.float32),
            pltpu.SemaphoreType.DMA,
        ],
    )
    def k(hl_hbm, src_hbm, dst_hbm, z_hbm, acc_out, si, di, rows, acc_sh, sem):
        cid = lax.axis_index("c")
        sid = lax.axis_index("s")
        wid = cid * NS + sid

        r0 = sid * RPW
        pltpu.sync_copy(z_hbm.at[pl.ds(r0, RPW), :], acc_sh.at[pl.ds(r0, RPW), :])
        plsc.subcore_barrier()

        base = wid * EPW

        @pl.loop(0, NCHUNK)
        def _(j):
            off = base + j * CH
            pltpu.sync_copy(src_hbm.at[pl.ds(off, CH)], si.at[0])
            pltpu.sync_copy(dst_hbm.at[pl.ds(off, CH)], di.at[0])
            pltpu.async_copy(hl_hbm.at[si.at[0]], rows, sem).wait()
            pltpu.sync_copy(rows, acc_sh.at[di.at[0]], add=True)

        plsc.subcore_barrier()
        pltpu.sync_copy(acc_sh.at[pl.ds(r0, RPW), :],
                        acc_out.at[cid, pl.ds(r0, RPW), :])

    return k(hl, src, dst, zeros16)


def _tc_layer1(x_pad, s1a, s1b, cnta, cntb, Wl1, bl1, Wr1, Wl2):
    CROWS = BR // 128

    def body(x_ref, sa_ref, sb_ref, ca_ref, cb_ref, wl1_ref, bl1_ref,
             wr1_ref, wl2_ref, h1_ref, hl_ref, inv_ref):
        cnt = ca_ref[...] + cb_ref[...]
        inv = 1.0 / jnp.maximum(cnt, 1.0)
        inv_ref[...] = inv
        s1 = (sa_ref[...] + sb_ref[...]) * inv.reshape(BR, 1)
        h1 = s1 @ wl1_ref[...] + bl1_ref[...] + x_ref[...] @ wr1_ref[...]
        h1 = jnp.maximum(h1, 0.0)
        h1_ref[...] = h1
        hl_ref[...] = h1 @ wl2_ref[...]

    row_spec = pl.BlockSpec((BR, D), lambda i: (i, 0))
    cnt_spec = pl.BlockSpec((CROWS, 128), lambda i: (i, 0))
    full = lambda shape: pl.BlockSpec(shape, lambda i: tuple(0 for _ in shape))
    return pl.pallas_call(
        body,
        grid=(GRID,),
        in_specs=[row_spec, row_spec, row_spec, cnt_spec, cnt_spec,
                  full((D, D)), full((1, D)), full((D, D)), full((D, NCLS))],
        out_specs=[row_spec, pl.BlockSpec((BR, NCLS), lambda i: (i, 0)),
                   cnt_spec],
        out_shape=[
            jax.ShapeDtypeStruct((NPAD, D), jnp.float32),
            jax.ShapeDtypeStruct((NPAD, NCLS), jnp.float32),
            jax.ShapeDtypeStruct((NPAD // 128, 128), jnp.float32),
        ],
    )(x_pad, s1a, s1b, cnta, cntb, Wl1, bl1.reshape(1, D), Wr1, Wl2)


def _tc_layer2(h1, s2a, s2b, inv, Wr2, bl2, batch2d):
    CROWS = BR // 128

    def body(h1_ref, sa_ref, sb_ref, inv_ref, wr2_ref, bl2_ref, b_ref,
             out_ref, psum, pcnt):
        i = pl.program_id(0)

        @pl.when(i == 0)
        def _():
            psum[...] = jnp.zeros_like(psum)
            pcnt[...] = jnp.zeros_like(pcnt)

        inv = inv_ref[...].reshape(BR, 1)
        h2 = (sa_ref[...] + sb_ref[...]) * inv + bl2_ref[...] \
            + h1_ref[...] @ wr2_ref[...]
        seg = b_ref[...].reshape(1, BR)
        oh = (lax.broadcasted_iota(jnp.int32, (N_GRAPHS, BR), 0) == seg)
        oh = oh.astype(jnp.float32)
        psum[...] += oh @ h2
        pcnt[...] += oh @ jnp.ones((BR, NCLS), jnp.float32)

        @pl.when(i == GRID - 1)
        def _():
            p = psum[...] / jnp.maximum(pcnt[...], 1.0)
            m = jnp.max(p, axis=1, keepdims=True)
            e = jnp.exp(p - m)
            lse = jnp.log(jnp.sum(e, axis=1, keepdims=True))
            out_ref[...] = p - m - lse

    row_spec = pl.BlockSpec((BR, D), lambda i: (i, 0))
    nar_spec = pl.BlockSpec((BR, NCLS), lambda i: (i, 0))
    cnt_spec = pl.BlockSpec((CROWS, 128), lambda i: (i, 0))
    full = lambda shape: pl.BlockSpec(shape, lambda i: tuple(0 for _ in shape))
    return pl.pallas_call(
        body,
        grid=(GRID,),
        in_specs=[row_spec, nar_spec, nar_spec, cnt_spec,
                  full((D, NCLS)), full((1, NCLS)), cnt_spec],
        out_specs=full((N_GRAPHS, NCLS)),
        out_shape=jax.ShapeDtypeStruct((N_GRAPHS, NCLS), jnp.float32),
        scratch_shapes=[
            pltpu.VMEM((N_GRAPHS, NCLS), jnp.float32),
            pltpu.VMEM((N_GRAPHS, NCLS), jnp.float32),
        ],
    )(h1, s2a, s2b, inv, Wr2, bl2.reshape(1, NCLS), batch2d)


def kernel(x, edge_index, batch, Wl1, bl1, Wr1, Wl2, bl2, Wr2):
    src = edge_index[0]
    dst = edge_index[1]
    zeros2d = jnp.zeros((NPAD, D), jnp.float32)
    zeros1d = jnp.zeros((NPAD,), jnp.float32)
    zeros16 = jnp.zeros((NPAD, NCLS), jnp.float32)

    accs, cnts = _sc_agg_wide(x, src, dst, zeros2d, zeros1d)

    x_pad = jnp.concatenate([x, jnp.zeros((NPAD - N_NODES, D), jnp.float32)])
    h1, hl, inv = _tc_layer1(
        x_pad, accs[0], accs[1],
        cnts[0].reshape(NPAD // 128, 128), cnts[1].reshape(NPAD // 128, 128),
        Wl1, bl1, Wr1, Wl2)

    s2 = _sc_agg_narrow(hl, src, dst, zeros16)

    batch2d = jnp.concatenate(
        [batch, jnp.full((NPAD - N_NODES,), N_GRAPHS, jnp.int32)]
    ).reshape(NPAD // 128, 128)
    return _tc_layer2(h1, s2[0], s2[1], inv, Wr2, bl2, batch2d)


# speedup vs baseline: 6.0680x; 6.0680x over previous
import functools

import jax
import jax.numpy as jnp
from jax import lax
from jax.experimental import pallas as pl
from jax.experimental.pallas import tpu as pltpu
from jax.experimental.pallas import tpu_sc as plsc

N_NODES = 10000
N_EDGES = 320000
D = 128
NCLS = 16
N_GRAPHS = 64

NC, NS = 2, 16
NW = NC * NS
NPAD = 10240
RPW = NPAD // NS
EPW = N_EDGES // NW
CH = 80
NCHUNK = EPW // CH

BR = 1024
GRID = NPAD // BR


def _sc_agg_wide(x, src, dst, zeros2d, zeros16, ones16):
    mesh = plsc.VectorSubcoreMesh(core_axis_name="c", subcore_axis_name="s")

    @functools.partial(
        pl.kernel,
        out_type=(
            jax.ShapeDtypeStruct((NC, NPAD, D), jnp.float32),
            jax.ShapeDtypeStruct((NC, NPAD, NCLS), jnp.float32),
        ),
        mesh=mesh,
        compiler_params=pltpu.CompilerParams(use_tc_tiling_on_sc=False),
        scratch_types=[
            pltpu.VMEM((1, CH), jnp.int32),
            pltpu.VMEM((1, CH), jnp.int32),
            pltpu.VMEM((CH, D), jnp.float32),
            pltpu.VMEM((CH, NCLS), jnp.float32),
            pltpu.VMEM_SHARED((NPAD, D), jnp.float32),
            pltpu.VMEM_SHARED((NPAD, NCLS), jnp.float32),
            pltpu.SemaphoreType.DMA,
        ],
    )
    def k(x_hbm, src_hbm, dst_hbm, z2_hbm, z16_hbm, ones_hbm, acc_out, cnt_out,
          si, di, rows, ones, acc_sh, cnt_sh, sem):
        cid = lax.axis_index("c")
        sid = lax.axis_index("s")
        wid = cid * NS + sid

        r0 = sid * RPW
        pltpu.sync_copy(z2_hbm.at[pl.ds(r0, RPW), :], acc_sh.at[pl.ds(r0, RPW), :])
        pltpu.sync_copy(z16_hbm.at[pl.ds(r0, RPW), :], cnt_sh.at[pl.ds(r0, RPW), :])
        pltpu.sync_copy(ones_hbm, ones)

        plsc.subcore_barrier()

        base = wid * EPW

        @pl.loop(0, NCHUNK)
        def _(j):
            off = base + j * CH
            pltpu.sync_copy(src_hbm.at[pl.ds(off, CH)], si.at[0])
            pltpu.sync_copy(dst_hbm.at[pl.ds(off, CH)], di.at[0])
            pltpu.async_copy(x_hbm.at[si.at[0]], rows, sem).wait()
            pltpu.sync_copy(rows, acc_sh.at[di.at[0]], add=True)
            pltpu.sync_copy(ones, cnt_sh.at[di.at[0]], add=True)

        plsc.subcore_barrier()

        pltpu.sync_copy(acc_sh.at[pl.ds(r0, RPW), :],
                        acc_out.at[cid, pl.ds(r0, RPW), :])
        pltpu.sync_copy(cnt_sh.at[pl.ds(r0, RPW), :],
                        cnt_out.at[cid, pl.ds(r0, RPW), :])

    return k(x, src, dst, zeros2d, zeros16, ones16)


def _sc_agg_narrow(hl, src, dst, zeros16):
    mesh = plsc.VectorSubcoreMesh(core_axis_name="c", subcore_axis_name="s")

    @functools.partial(
        pl.kernel,
        out_type=jax.ShapeDtypeStruct((NC, NPAD, NCLS), jnp.float32),
        mesh=mesh,
        compiler_params=pltpu.CompilerParams(use_tc_tiling_on_sc=False),
        scratch_types=[
            pltpu.VMEM((1, CH), jnp.int32),
            pltpu.VMEM((1, CH), jnp.int32),
            pltpu.VMEM((CH, NCLS), jnp.float32),
            pltpu.VMEM_SHARED((NPAD, NCLS), jnp.float32),
            pltpu.SemaphoreType.DMA,
        ],
    )
    def k(hl_hbm, src_hbm, dst_hbm, z_hbm, acc_out, si, di, rows, acc_sh, sem):
        cid = lax.axis_index("c")
        sid = lax.axis_index("s")
        wid = cid * NS + sid

        r0 = sid * RPW
        pltpu.sync_copy(z_hbm.at[pl.ds(r0, RPW), :], acc_sh.at[pl.ds(r0, RPW), :])
        plsc.subcore_barrier()

        base = wid * EPW

        @pl.loop(0, NCHUNK)
        def _(j):
            off = base + j * CH
            pltpu.sync_copy(src_hbm.at[pl.ds(off, CH)], si.at[0])
            pltpu.sync_copy(dst_hbm.at[pl.ds(off, CH)], di.at[0])
            pltpu.async_copy(hl_hbm.at[si.at[0]], rows, sem).wait()
            pltpu.sync_copy(rows, acc_sh.at[di.at[0]], add=True)

        plsc.subcore_barrier()
        pltpu.sync_copy(acc_sh.at[pl.ds(r0, RPW), :],
                        acc_out.at[cid, pl.ds(r0, RPW), :])

    return k(hl, src, dst, zeros16)


def _tc_layer1(x_pad, s1a, s1b, cnta, cntb, Wl1, bl1, Wr1, Wl2):
    def body(x_ref, sa_ref, sb_ref, ca_ref, cb_ref, wl1_ref, bl1_ref,
             wr1_ref, wl2_ref, h1_ref, hl_ref, inv_ref):
        cnt = ca_ref[:, 0:1] + cb_ref[:, 0:1]
        inv = 1.0 / jnp.maximum(cnt, 1.0)
        inv_ref[...] = inv
        s1 = (sa_ref[...] + sb_ref[...]) * inv
        h1 = s1 @ wl1_ref[...] + bl1_ref[...] + x_ref[...] @ wr1_ref[...]
        h1 = jnp.maximum(h1, 0.0)
        h1_ref[...] = h1
        hl_ref[...] = h1 @ wl2_ref[...]

    row_spec = pl.BlockSpec((BR, D), lambda i: (i, 0))
    nar_spec = pl.BlockSpec((BR, NCLS), lambda i: (i, 0))
    col_spec = pl.BlockSpec((BR, 1), lambda i: (i, 0))
    full = lambda shape: pl.BlockSpec(shape, lambda i: tuple(0 for _ in shape))
    return pl.pallas_call(
        body,
        grid=(GRID,),
        in_specs=[row_spec, row_spec, row_spec, nar_spec, nar_spec,
                  full((D, D)), full((1, D)), full((D, D)), full((D, NCLS))],
        out_specs=[row_spec, pl.BlockSpec((BR, NCLS), lambda i: (i, 0)),
                   col_spec],
        out_shape=[
            jax.ShapeDtypeStruct((NPAD, D), jnp.float32),
            jax.ShapeDtypeStruct((NPAD, NCLS), jnp.float32),
            jax.ShapeDtypeStruct((NPAD, 1), jnp.float32),
        ],
    )(x_pad, s1a, s1b, cnta, cntb, Wl1, bl1.reshape(1, D), Wr1, Wl2)


def _tc_layer2(h1, s2a, s2b, inv, Wr2, bl2, batch_col):
    def body(h1_ref, sa_ref, sb_ref, inv_ref, wr2_ref, bl2_ref, b_ref,
             out_ref, psum, pcnt):
        i = pl.program_id(0)

        @pl.when(i == 0)
        def _():
            psum[...] = jnp.zeros_like(psum)
            pcnt[...] = jnp.zeros_like(pcnt)

        h2 = (sa_ref[...] + sb_ref[...]) * inv_ref[...] + bl2_ref[...] \
            + h1_ref[...] @ wr2_ref[...]
        oh = (b_ref[...] == lax.broadcasted_iota(jnp.int32, (1, N_GRAPHS), 1))
        oh = oh.astype(jnp.float32)
        dn = (((0,), (0,)), ((), ()))
        psum[...] += lax.dot_general(oh, h2, dn,
                                     preferred_element_type=jnp.float32)
        pcnt[...] += lax.dot_general(oh, jnp.ones((BR, NCLS), jnp.float32), dn,
                                     preferred_element_type=jnp.float32)

        @pl.when(i == GRID - 1)
        def _():
            p = psum[...] / jnp.maximum(pcnt[...], 1.0)
            m = jnp.max(p, axis=1, keepdims=True)
            e = jnp.exp(p - m)
            lse = jnp.log(jnp.sum(e, axis=1, keepdims=True))
            out_ref[...] = p - m - lse

    row_spec = pl.BlockSpec((BR, D), lambda i: (i, 0))
    nar_spec = pl.BlockSpec((BR, NCLS), lambda i: (i, 0))
    col_spec = pl.BlockSpec((BR, 1), lambda i: (i, 0))
    full = lambda shape: pl.BlockSpec(shape, lambda i: tuple(0 for _ in shape))
    return pl.pallas_call(
        body,
        grid=(GRID,),
        in_specs=[row_spec, nar_spec, nar_spec, col_spec,
                  full((D, NCLS)), full((1, NCLS)), col_spec],
        out_specs=full((N_GRAPHS, NCLS)),
        out_shape=jax.ShapeDtypeStruct((N_GRAPHS, NCLS), jnp.float32),
        scratch_shapes=[
            pltpu.VMEM((N_GRAPHS, NCLS), jnp.float32),
            pltpu.VMEM((N_GRAPHS, NCLS), jnp.float32),
        ],
    )(h1, s2a, s2b, inv, Wr2, bl2.reshape(1, NCLS), batch_col)


def kernel(x, edge_index, batch, Wl1, bl1, Wr1, Wl2, bl2, Wr2):
    src = edge_index[0]
    dst = edge_index[1]
    zeros2d = jnp.zeros((NPAD, D), jnp.float32)
    zeros16 = jnp.zeros((NPAD, NCLS), jnp.float32)
    ones16 = jnp.ones((CH, NCLS), jnp.float32)

    accs, cnts = _sc_agg_wide(x, src, dst, zeros2d, zeros16, ones16)

    x_pad = jnp.concatenate([x, jnp.zeros((NPAD - N_NODES, D), jnp.float32)])
    h1, hl, inv = _tc_layer1(
        x_pad, accs[0], accs[1], cnts[0], cnts[1],
        Wl1, bl1, Wr1, Wl2)

    s2 = _sc_agg_narrow(hl, src, dst, zeros16)

    batch_col = jnp.concatenate(
        [batch, jnp.full((NPAD - N_NODES,), N_GRAPHS, jnp.int32)]
    ).reshape(NPAD, 1)
    return _tc_layer2(h1, s2[0], s2[1], inv, Wr2, bl2, batch_col)


# speedup vs baseline: 6.0919x; 1.0039x over previous
import functools

import jax
import jax.numpy as jnp
from jax import lax
from jax.experimental import pallas as pl
from jax.experimental.pallas import tpu as pltpu
from jax.experimental.pallas import tpu_sc as plsc

N_NODES = 10000
N_EDGES = 320000
D = 128
NCLS = 16
N_GRAPHS = 64

NC, NS = 2, 16
NW = NC * NS
NPAD = 10240
RPW = NPAD // NS
EP = 327680
CHW = 64
NTW = EP // (NW * CHW)
CHN = 128
NTN = EP // (NW * CHN)

BR = 1024
GRID = NPAD // BR


def _sc_agg_wide(x, src, dst, zeros2d, zeros16, ones16):
    mesh = plsc.VectorSubcoreMesh(core_axis_name="c", subcore_axis_name="s")

    @functools.partial(
        pl.kernel,
        out_type=(
            jax.ShapeDtypeStruct((NC, NPAD, D), jnp.float32),
            jax.ShapeDtypeStruct((NC, NPAD, NCLS), jnp.float32),
        ),
        mesh=mesh,
        compiler_params=pltpu.CompilerParams(use_tc_tiling_on_sc=False),
        scratch_types=[
            pltpu.VMEM((NTW, CHW), jnp.int32),
            pltpu.VMEM((NTW, CHW), jnp.int32),
            pltpu.VMEM((CHW, D), jnp.float32),
            pltpu.VMEM((CHW, D), jnp.float32),
            pltpu.VMEM((CHW, NCLS), jnp.float32),
            pltpu.VMEM_SHARED((NPAD, D), jnp.float32),
            pltpu.VMEM_SHARED((NPAD, NCLS), jnp.float32),
            pltpu.SemaphoreType.DMA,
            pltpu.SemaphoreType.DMA,
        ],
    )
    def k(x_hbm, src_hbm, dst_hbm, z2_hbm, z16_hbm, ones_hbm, acc_out, cnt_out,
          sib, dib, rows_a, rows_b, ones, acc_sh, cnt_sh, sem_a, sem_b):
        cid = lax.axis_index("c")
        sid = lax.axis_index("s")
        wid = cid * NS + sid

        r0 = sid * RPW
        pltpu.sync_copy(z2_hbm.at[pl.ds(r0, RPW), :], acc_sh.at[pl.ds(r0, RPW), :])
        pltpu.sync_copy(z16_hbm.at[pl.ds(r0, RPW), :], cnt_sh.at[pl.ds(r0, RPW), :])
        pltpu.sync_copy(ones_hbm, ones)

        w0 = wid * NTW
        pltpu.sync_copy(src_hbm.at[pl.ds(w0, NTW), :], sib)
        pltpu.sync_copy(dst_hbm.at[pl.ds(w0, NTW), :], dib)

        plsc.subcore_barrier()

        pltpu.async_copy(x_hbm.at[sib.at[0]], rows_a, sem_a)
        pltpu.async_copy(x_hbm.at[sib.at[1]], rows_b, sem_b)

        @pl.loop(0, NTW - 2, step=2)
        def _(t):
            pltpu.make_async_copy(x_hbm.at[sib.at[t]], rows_a, sem_a).wait()
            pltpu.sync_copy(rows_a, acc_sh.at[dib.at[t]], add=True)
            pltpu.sync_copy(ones, cnt_sh.at[dib.at[t]], add=True)
            pltpu.async_copy(x_hbm.at[sib.at[t + 2]], rows_a, sem_a)
            pltpu.make_async_copy(x_hbm.at[sib.at[t + 1]], rows_b, sem_b).wait()
            pltpu.sync_copy(rows_b, acc_sh.at[dib.at[t + 1]], add=True)
            pltpu.sync_copy(ones, cnt_sh.at[dib.at[t + 1]], add=True)
            pltpu.async_copy(x_hbm.at[sib.at[t + 3]], rows_b, sem_b)

        pltpu.make_async_copy(x_hbm.at[sib.at[NTW - 2]], rows_a, sem_a).wait()
        pltpu.sync_copy(rows_a, acc_sh.at[dib.at[NTW - 2]], add=True)
        pltpu.sync_copy(ones, cnt_sh.at[dib.at[NTW - 2]], add=True)
        pltpu.make_async_copy(x_hbm.at[sib.at[NTW - 1]], rows_b, sem_b).wait()
        pltpu.sync_copy(rows_b, acc_sh.at[dib.at[NTW - 1]], add=True)
        pltpu.sync_copy(ones, cnt_sh.at[dib.at[NTW - 1]], add=True)

        plsc.subcore_barrier()

        pltpu.sync_copy(acc_sh.at[pl.ds(r0, RPW), :],
                        acc_out.at[cid, pl.ds(r0, RPW), :])
        pltpu.sync_copy(cnt_sh.at[pl.ds(r0, RPW), :],
                        cnt_out.at[cid, pl.ds(r0, RPW), :])

    return k(x, src, dst, zeros2d, zeros16, ones16)


def _sc_agg_narrow(hl, src, dst, zeros16):
    mesh = plsc.VectorSubcoreMesh(core_axis_name="c", subcore_axis_name="s")

    @functools.partial(
        pl.kernel,
        out_type=jax.ShapeDtypeStruct((NC, NPAD, NCLS), jnp.float32),
        mesh=mesh,
        compiler_params=pltpu.CompilerParams(use_tc_tiling_on_sc=False),
        scratch_types=[
            pltpu.VMEM((NTN, CHN), jnp.int32),
            pltpu.VMEM((NTN, CHN), jnp.int32),
            pltpu.VMEM((CHN, NCLS), jnp.float32),
            pltpu.VMEM((CHN, NCLS), jnp.float32),
            pltpu.VMEM_SHARED((NPAD, NCLS), jnp.float32),
            pltpu.SemaphoreType.DMA,
            pltpu.SemaphoreType.DMA,
        ],
    )
    def k(hl_hbm, src_hbm, dst_hbm, z_hbm, acc_out,
          sib, dib, rows_a, rows_b, acc_sh, sem_a, sem_b):
        cid = lax.axis_index("c")
        sid = lax.axis_index("s")
        wid = cid * NS + sid

        r0 = sid * RPW
        pltpu.sync_copy(z_hbm.at[pl.ds(r0, RPW), :], acc_sh.at[pl.ds(r0, RPW), :])

        w0 = wid * NTN
        pltpu.sync_copy(src_hbm.at[pl.ds(w0, NTN), :], sib)
        pltpu.sync_copy(dst_hbm.at[pl.ds(w0, NTN), :], dib)

        plsc.subcore_barrier()

        pltpu.async_copy(hl_hbm.at[sib.at[0]], rows_a, sem_a)
        pltpu.async_copy(hl_hbm.at[sib.at[1]], rows_b, sem_b)

        @pl.loop(0, NTN - 2, step=2)
        def _(t):
            pltpu.make_async_copy(hl_hbm.at[sib.at[t]], rows_a, sem_a).wait()
            pltpu.sync_copy(rows_a, acc_sh.at[dib.at[t]], add=True)
            pltpu.async_copy(hl_hbm.at[sib.at[t + 2]], rows_a, sem_a)
            pltpu.make_async_copy(hl_hbm.at[sib.at[t + 1]], rows_b, sem_b).wait()
            pltpu.sync_copy(rows_b, acc_sh.at[dib.at[t + 1]], add=True)
            pltpu.async_copy(hl_hbm.at[sib.at[t + 3]], rows_b, sem_b)

        pltpu.make_async_copy(hl_hbm.at[sib.at[NTN - 2]], rows_a, sem_a).wait()
        pltpu.sync_copy(rows_a, acc_sh.at[dib.at[NTN - 2]], add=True)
        pltpu.make_async_copy(hl_hbm.at[sib.at[NTN - 1]], rows_b, sem_b).wait()
        pltpu.sync_copy(rows_b, acc_sh.at[dib.at[NTN - 1]], add=True)

        plsc.subcore_barrier()
        pltpu.sync_copy(acc_sh.at[pl.ds(r0, RPW), :],
                        acc_out.at[cid, pl.ds(r0, RPW), :])

    return k(hl, src, dst, zeros16)


def _tc_layer1(x_pad, s1a, s1b, cnta, cntb, Wl1, bl1, Wr1, Wl2):
    def body(x_ref, sa_ref, sb_ref, ca_ref, cb_ref, wl1_ref, bl1_ref,
             wr1_ref, wl2_ref, h1_ref, hl_ref, inv_ref):
        cnt = ca_ref[:, 0:1] + cb_ref[:, 0:1]
        inv = 1.0 / jnp.maximum(cnt, 1.0)
        inv_ref[...] = inv
        s1 = (sa_ref[...] + sb_ref[...]) * inv
        h1 = s1 @ wl1_ref[...] + bl1_ref[...] + x_ref[...] @ wr1_ref[...]
        h1 = jnp.maximum(h1, 0.0)
        h1_ref[...] = h1
        hl_ref[...] = h1 @ wl2_ref[...]

    row_spec = pl.BlockSpec((BR, D), lambda i: (i, 0))
    nar_spec = pl.BlockSpec((BR, NCLS), lambda i: (i, 0))
    col_spec = pl.BlockSpec((BR, 1), lambda i: (i, 0))
    full = lambda shape: pl.BlockSpec(shape, lambda i: tuple(0 for _ in shape))
    return pl.pallas_call(
        body,
        grid=(GRID,),
        in_specs=[row_spec, row_spec, row_spec, nar_spec, nar_spec,
                  full((D, D)), full((1, D)), full((D, D)), full((D, NCLS))],
        out_specs=[row_spec, pl.BlockSpec((BR, NCLS), lambda i: (i, 0)),
                   col_spec],
        out_shape=[
            jax.ShapeDtypeStruct((NPAD, D), jnp.float32),
            jax.ShapeDtypeStruct((NPAD, NCLS), jnp.float32),
            jax.ShapeDtypeStruct((NPAD, 1), jnp.float32),
        ],
    )(x_pad, s1a, s1b, cnta, cntb, Wl1, bl1.reshape(1, D), Wr1, Wl2)


def _tc_layer2(h1, s2a, s2b, inv, Wr2, bl2, batch_col):
    def body(h1_ref, sa_ref, sb_ref, inv_ref, wr2_ref, bl2_ref, b_ref,
             out_ref, psum, pcnt):
        i = pl.program_id(0)

        @pl.when(i == 0)
        def _():
            psum[...] = jnp.zeros_like(psum)
            pcnt[...] = jnp.zeros_like(pcnt)

        h2 = (sa_ref[...] + sb_ref[...]) * inv_ref[...] + bl2_ref[...] \
            + h1_ref[...] @ wr2_ref[...]
        oh = (b_ref[...] == lax.broadcasted_iota(jnp.int32, (1, N_GRAPHS), 1))
        oh = oh.astype(jnp.float32)
        dn = (((0,), (0,)), ((), ()))
        psum[...] += lax.dot_general(oh, h2, dn,
                                     preferred_element_type=jnp.float32)
        pcnt[...] += lax.dot_general(oh, jnp.ones((BR, NCLS), jnp.float32), dn,
                                     preferred_element_type=jnp.float32)

        @pl.when(i == GRID - 1)
        def _():
            p = psum[...] / jnp.maximum(pcnt[...], 1.0)
            m = jnp.max(p, axis=1, keepdims=True)
            e = jnp.exp(p - m)
            lse = jnp.log(jnp.sum(e, axis=1, keepdims=True))
            out_ref[...] = p - m - lse

    row_spec = pl.BlockSpec((BR, D), lambda i: (i, 0))
    nar_spec = pl.BlockSpec((BR, NCLS), lambda i: (i, 0))
    col_spec = pl.BlockSpec((BR, 1), lambda i: (i, 0))
    full = lambda shape: pl.BlockSpec(shape, lambda i: tuple(0 for _ in shape))
    return pl.pallas_call(
        body,
        grid=(GRID,),
        in_specs=[row_spec, nar_spec, nar_spec, col_spec,
                  full((D, NCLS)), full((1, NCLS)), col_spec],
        out_specs=full((N_GRAPHS, NCLS)),
        out_shape=jax.ShapeDtypeStruct((N_GRAPHS, NCLS), jnp.float32),
        scratch_shapes=[
            pltpu.VMEM((N_GRAPHS, NCLS), jnp.float32),
            pltpu.VMEM((N_GRAPHS, NCLS), jnp.float32),
        ],
    )(h1, s2a, s2b, inv, Wr2, bl2.reshape(1, NCLS), batch_col)


def kernel(x, edge_index, batch, Wl1, bl1, Wr1, Wl2, bl2, Wr2):
    npad_e = EP - N_EDGES
    pad_dst = (N_NODES + jnp.arange(npad_e, dtype=jnp.int32)
               % (NPAD - N_NODES))
    src = jnp.concatenate([edge_index[0], jnp.zeros((npad_e,), jnp.int32)])
    dst = jnp.concatenate([edge_index[1], pad_dst])
    zeros2d = jnp.zeros((NPAD, D), jnp.float32)
    zeros16 = jnp.zeros((NPAD, NCLS), jnp.float32)
    ones16 = jnp.ones((CHW, NCLS), jnp.float32)

    accs, cnts = _sc_agg_wide(x, src.reshape(-1, CHW),
                              dst.reshape(-1, CHW),
                              zeros2d, zeros16, ones16)

    x_pad = jnp.concatenate([x, jnp.zeros((NPAD - N_NODES, D), jnp.float32)])
    h1, hl, inv = _tc_layer1(
        x_pad, accs[0], accs[1], cnts[0], cnts[1],
        Wl1, bl1, Wr1, Wl2)

    s2 = _sc_agg_narrow(hl, src.reshape(-1, CHN),
                        dst.reshape(-1, CHN), zeros16)

    batch_col = jnp.concatenate(
        [batch, jnp.full((NPAD - N_NODES,), N_GRAPHS, jnp.int32)]
    ).reshape(NPAD, 1)
    return _tc_layer2(h1, s2[0], s2[1], inv, Wr2, bl2, batch_col)
